# baseline (speedup 1.0000x reference)
import functools

import jax
import jax.numpy as jnp
from jax.experimental import pallas as pl

NUM_EMBS = 8192
EMB_DIM = 32
BETA = 0.25
N_TOKENS = 8192
TILE = 512
GRID = N_TOKENS // TILE


def _body(x_ref, sx_ref, sw_ref, w_ref, wneg2_ref,
          idx_ref, zq_ref, hist_ref, loss_ref, perp_ref):
    step = pl.program_id(0)

    @pl.when(step == 0)
    def _init():
        hist_ref[...] = jnp.zeros_like(hist_ref)
        loss_ref[...] = jnp.zeros_like(loss_ref)
        perp_ref[...] = jnp.zeros_like(perp_ref)

    x = x_ref[...]
    w = w_ref[...]

    mm2 = jax.lax.dot_general(x, wneg2_ref[...], (((1,), (1,)), ((), ())),
                              preferred_element_type=jnp.float32)
    d = (sx_ref[...] + sw_ref[...]) + mm2

    gmin = jnp.min(d, axis=1, keepdims=True)
    col = jax.lax.broadcasted_iota(jnp.int32, (TILE, NUM_EMBS), 1)
    idx = jnp.min(jnp.where(d == gmin, col, NUM_EMBS), axis=1,
                  keepdims=True)
    idx_ref[...] = idx

    oh = (col == idx).astype(jnp.float32)
    q = jax.lax.dot_general(oh, w, (((1,), (0,)), ((), ())),
                            preferred_element_type=jnp.float32)

    hist_ref[...] += jnp.sum(oh, axis=0)[None, :]
    zq_ref[...] = x + (q - x)
    loss_ref[...] += jnp.sum((q - x) ** 2)

    @pl.when(step == GRID - 1)
    def _fini():
        loss_ref[...] = (1.0 + BETA) * loss_ref[...] / (N_TOKENS * EMB_DIM)
        probs = hist_ref[...] / N_TOKENS
        ent = -jnp.sum(probs * jnp.log(probs + 1e-10))
        perp_ref[...] = jnp.exp(ent) * jnp.ones_like(perp_ref)


def kernel(z_e_x, W):
    B, C, H, Wd = z_e_x.shape
    x_flat = jnp.transpose(z_e_x, (0, 2, 3, 1)).reshape(-1, EMB_DIM)
    sx = jnp.sum(x_flat ** 2, axis=1, keepdims=True)
    sw = jnp.sum(W ** 2, axis=1)[None, :]

    idx, zq, hist, loss, perp = pl.pallas_call(
        _body,
        grid=(GRID,),
        in_specs=[
            pl.BlockSpec((TILE, EMB_DIM), lambda i: (i, 0)),
            pl.BlockSpec((TILE, 1), lambda i: (i, 0)),
            pl.BlockSpec((1, NUM_EMBS), lambda i: (0, 0)),
            pl.BlockSpec((NUM_EMBS, EMB_DIM), lambda i: (0, 0)),
            pl.BlockSpec((NUM_EMBS, EMB_DIM), lambda i: (0, 0)),
        ],
        out_specs=[
            pl.BlockSpec((TILE, 1), lambda i: (i, 0)),
            pl.BlockSpec((TILE, EMB_DIM), lambda i: (i, 0)),
            pl.BlockSpec((1, NUM_EMBS), lambda i: (0, 0)),
            pl.BlockSpec((1, 1), lambda i: (0, 0)),
            pl.BlockSpec((1, 1), lambda i: (0, 0)),
        ],
        out_shape=[
            jax.ShapeDtypeStruct((N_TOKENS, 1), jnp.int32),
            jax.ShapeDtypeStruct((N_TOKENS, EMB_DIM), jnp.float32),
            jax.ShapeDtypeStruct((1, NUM_EMBS), jnp.float32),
            jax.ShapeDtypeStruct((1, 1), jnp.float32),
            jax.ShapeDtypeStruct((1, 1), jnp.float32),
        ],
    )(x_flat, sx, sw, W, -2.0 * W)

    z_q_x = jnp.transpose(zq.reshape(B, H, Wd, C), (0, 3, 1, 2))
    return (loss[0, 0], z_q_x, perp[0, 0], idx)


# speedup vs baseline: 1.1942x; 1.1942x over previous
import jax
import jax.numpy as jnp
from jax.experimental import pallas as pl

NUM_EMBS = 8192
EMB_DIM = 32
BETA = 0.25
N_TOKENS = 8192
TILE = 512
GRID = N_TOKENS // TILE


def _body(x_ref, sx_ref, swr_ref, wr_ref, wneg2r_ref,
          idx_ref, zq_ref, hist_ref, loss_ref, perp_ref):
    step = pl.program_id(0)

    @pl.when(step == 0)
    def _init():
        hist_ref[...] = jnp.zeros_like(hist_ref)
        loss_ref[...] = jnp.zeros_like(loss_ref)
        perp_ref[...] = jnp.zeros_like(perp_ref)

    x = x_ref[...]

    mm2 = jax.lax.dot_general(x, wneg2r_ref[...], (((1,), (1,)), ((), ())),
                              preferred_element_type=jnp.float32)
    d = (sx_ref[...] + swr_ref[...]) + mm2

    am = jnp.argmin(d, axis=1).astype(jnp.int32)[:, None]
    idx_ref[...] = (127 - (am & 127)) * 64 + (am >> 7)

    col = jax.lax.broadcasted_iota(jnp.int32, (TILE, NUM_EMBS), 1)
    oh = (col == am).astype(jnp.float32)
    q = jax.lax.dot_general(oh, wr_ref[...], (((1,), (0,)), ((), ())),
                            preferred_element_type=jnp.float32)

    hist_ref[...] += jnp.sum(oh, axis=0)[None, :]
    zq_ref[...] = x + (q - x)
    loss_ref[...] += jnp.sum((q - x) ** 2)

    @pl.when(step == GRID - 1)
    def _fini():
        loss_ref[...] = (1.0 + BETA) * loss_ref[...] / (N_TOKENS * EMB_DIM)
        probs = hist_ref[...] / N_TOKENS
        ent = -jnp.sum(probs * jnp.log(probs + 1e-10))
        perp_ref[...] = jnp.exp(ent) * jnp.ones_like(perp_ref)


def kernel(z_e_x, W):
    B, C, H, Wd = z_e_x.shape
    x_flat = jnp.transpose(z_e_x, (0, 2, 3, 1)).reshape(-1, EMB_DIM)
    sx = jnp.sum(x_flat ** 2, axis=1, keepdims=True)
    sw = jnp.sum(W ** 2, axis=1)
    p = jnp.arange(NUM_EMBS, dtype=jnp.int32)
    inv = (127 - (p % 128)) * 64 + (p // 128)
    w_rev = W[inv]
    swr = sw[inv][None, :]
    wneg2r = -2.0 * w_rev

    idx, zq, hist, loss, perp = pl.pallas_call(
        _body,
        grid=(GRID,),
        in_specs=[
            pl.BlockSpec((TILE, EMB_DIM), lambda i: (i, 0)),
            pl.BlockSpec((TILE, 1), lambda i: (i, 0)),
            pl.BlockSpec((1, NUM_EMBS), lambda i: (0, 0)),
            pl.BlockSpec((NUM_EMBS, EMB_DIM), lambda i: (0, 0)),
            pl.BlockSpec((NUM_EMBS, EMB_DIM), lambda i: (0, 0)),
        ],
        out_specs=[
            pl.BlockSpec((TILE, 1), lambda i: (i, 0)),
            pl.BlockSpec((TILE, EMB_DIM), lambda i: (i, 0)),
            pl.BlockSpec((1, NUM_EMBS), lambda i: (0, 0)),
            pl.BlockSpec((1, 1), lambda i: (0, 0)),
            pl.BlockSpec((1, 1), lambda i: (0, 0)),
        ],
        out_shape=[
            jax.ShapeDtypeStruct((N_TOKENS, 1), jnp.int32),
            jax.ShapeDtypeStruct((N_TOKENS, EMB_DIM), jnp.float32),
            jax.ShapeDtypeStruct((1, NUM_EMBS), jnp.float32),
            jax.ShapeDtypeStruct((1, 1), jnp.float32),
            jax.ShapeDtypeStruct((1, 1), jnp.float32),
        ],
    )(x_flat, sx, swr, w_rev, wneg2r)

    z_q_x = jnp.transpose(zq.reshape(B, H, Wd, C), (0, 3, 1, 2))
    return (loss[0, 0], z_q_x, perp[0, 0], idx)


# speedup vs baseline: 1.2210x; 1.0225x over previous
import jax
import jax.numpy as jnp
from jax.experimental import pallas as pl

NUM_EMBS = 8192
EMB_DIM = 32
BETA = 0.25
N_TOKENS = 8192
TILE = 512
GRID = N_TOKENS // TILE


def _body(x_ref, sx_ref, swr_ref, wr_ref, wneg2r_ref,
          idx_ref, zq_ref, hist_ref, loss_ref, perp_ref):
    step = pl.program_id(0)

    @pl.when(step == 0)
    def _init():
        hist_ref[...] = jnp.zeros_like(hist_ref)
        loss_ref[...] = jnp.zeros_like(loss_ref)
        perp_ref[...] = jnp.zeros_like(perp_ref)

    x = x_ref[...]

    mm2 = jax.lax.dot_general(x, wneg2r_ref[...], (((1,), (1,)), ((), ())),
                              preferred_element_type=jnp.float32)
    d = (sx_ref[...] + swr_ref[...]) + mm2

    am = jnp.argmin(d, axis=1).astype(jnp.int32)[:, None]
    idx = (127 - (am & 127)) * 64 + (am >> 7)
    idx_ref[...] = idx

    col = jax.lax.broadcasted_iota(jnp.int32, (TILE, NUM_EMBS), 1)
    oh = (col == idx).astype(jnp.float32)
    q = jax.lax.dot_general(oh, wr_ref[...], (((1,), (0,)), ((), ())),
                            preferred_element_type=jnp.float32)

    hist_ref[...] += jnp.sum(oh, axis=0)[None, :]
    zq_ref[...] = x + (q - x)
    loss_ref[...] += jnp.sum((q - x) ** 2)

    @pl.when(step == GRID - 1)
    def _fini():
        loss_ref[...] = (1.0 + BETA) * loss_ref[...] / (N_TOKENS * EMB_DIM)
        probs = hist_ref[...] / N_TOKENS
        ent = -jnp.sum(probs * jnp.log(probs + 1e-10))
        perp_ref[...] = jnp.exp(ent) * jnp.ones_like(perp_ref)


def kernel(z_e_x, W):
    B, C, H, Wd = z_e_x.shape
    x_flat = jnp.transpose(z_e_x, (0, 2, 3, 1)).reshape(-1, EMB_DIM)
    sx = jnp.sum(x_flat ** 2, axis=1, keepdims=True)
    sw = jnp.sum(W ** 2, axis=1)
    w_rev = W
    wneg2r = (-2.0 * W).reshape(128, 64, EMB_DIM)[::-1]\
        .transpose(1, 0, 2).reshape(NUM_EMBS, EMB_DIM)
    swr = sw.reshape(128, 64)[::-1].T.reshape(1, NUM_EMBS)

    idx, zq, hist, loss, perp = pl.pallas_call(
        _body,
        grid=(GRID,),
        in_specs=[
            pl.BlockSpec((TILE, EMB_DIM), lambda i: (i, 0)),
            pl.BlockSpec((TILE, 1), lambda i: (i, 0)),
            pl.BlockSpec((1, NUM_EMBS), lambda i: (0, 0)),
            pl.BlockSpec((NUM_EMBS, EMB_DIM), lambda i: (0, 0)),
            pl.BlockSpec((NUM_EMBS, EMB_DIM), lambda i: (0, 0)),
        ],
        out_specs=[
            pl.BlockSpec((TILE, 1), lambda i: (i, 0)),
            pl.BlockSpec((TILE, EMB_DIM), lambda i: (i, 0)),
            pl.BlockSpec((1, NUM_EMBS), lambda i: (0, 0)),
            pl.BlockSpec((1, 1), lambda i: (0, 0)),
            pl.BlockSpec((1, 1), lambda i: (0, 0)),
        ],
        out_shape=[
            jax.ShapeDtypeStruct((N_TOKENS, 1), jnp.int32),
            jax.ShapeDtypeStruct((N_TOKENS, EMB_DIM), jnp.float32),
            jax.ShapeDtypeStruct((1, NUM_EMBS), jnp.float32),
            jax.ShapeDtypeStruct((1, 1), jnp.float32),
            jax.ShapeDtypeStruct((1, 1), jnp.float32),
        ],
    )(x_flat, sx, swr, w_rev, wneg2r)

    z_q_x = jnp.transpose(zq.reshape(B, H, Wd, C), (0, 3, 1, 2))
    return (loss[0, 0], z_q_x, perp[0, 0], idx)


# speedup vs baseline: 1.2288x; 1.0064x over previous
import jax
import jax.numpy as jnp
from jax.experimental import pallas as pl

NUM_EMBS = 8192
EMB_DIM = 32
BETA = 0.25
N_TOKENS = 8192
TILE = 512
GRID = N_TOKENS // TILE


def _body(x_ref, sx_ref, swr_ref, wr_ref, wneg2r_ref,
          idx_ref, zq_ref, hist_ref, loss_ref, perp_ref):
    step = pl.program_id(0)

    @pl.when(step == 0)
    def _init():
        hist_ref[...] = jnp.zeros_like(hist_ref)
        loss_ref[...] = jnp.zeros_like(loss_ref)
        perp_ref[...] = jnp.zeros_like(perp_ref)

    x = x_ref[...]

    mm2 = jax.lax.dot_general(x, wneg2r_ref[...], (((1,), (1,)), ((), ())),
                              preferred_element_type=jnp.float32)
    d = (sx_ref[...] + swr_ref[...]) + mm2

    am = jnp.argmin(d, axis=1).astype(jnp.int32)[:, None]
    idx = (127 - (am & 127)) * 64 + (am >> 7)
    idx_ref[...] = idx

    col = jax.lax.broadcasted_iota(jnp.int32, (TILE, NUM_EMBS), 1)
    oh = (col == idx).astype(jnp.float32)
    q = jax.lax.dot_general(oh, wr_ref[...], (((1,), (0,)), ((), ())),
                            preferred_element_type=jnp.float32)

    hist_ref[...] += jnp.sum(oh, axis=0)[None, :]
    zq = x + (q - x)
    zq_ref[...] = zq.T[None]
    loss_ref[...] += jnp.sum((q - x) ** 2)

    @pl.when(step == GRID - 1)
    def _fini():
        loss_ref[...] = (1.0 + BETA) * loss_ref[...] / (N_TOKENS * EMB_DIM)
        probs = hist_ref[...] / N_TOKENS
        ent = -jnp.sum(probs * jnp.log(probs + 1e-10))
        perp_ref[...] = jnp.exp(ent) * jnp.ones_like(perp_ref)


def kernel(z_e_x, W):
    B, C, H, Wd = z_e_x.shape
    x_flat = jnp.transpose(z_e_x, (0, 2, 3, 1)).reshape(-1, EMB_DIM)
    sx = jnp.sum(x_flat ** 2, axis=1, keepdims=True)
    sw = jnp.sum(W ** 2, axis=1)
    w_rev = W
    wneg2r = (-2.0 * W).reshape(128, 64, EMB_DIM)[::-1]\
        .transpose(1, 0, 2).reshape(NUM_EMBS, EMB_DIM)
    swr = sw.reshape(128, 64)[::-1].T.reshape(1, NUM_EMBS)

    idx, zq, hist, loss, perp = pl.pallas_call(
        _body,
        grid=(GRID,),
        in_specs=[
            pl.BlockSpec((TILE, EMB_DIM), lambda i: (i, 0)),
            pl.BlockSpec((TILE, 1), lambda i: (i, 0)),
            pl.BlockSpec((1, NUM_EMBS), lambda i: (0, 0)),
            pl.BlockSpec((NUM_EMBS, EMB_DIM), lambda i: (0, 0)),
            pl.BlockSpec((NUM_EMBS, EMB_DIM), lambda i: (0, 0)),
        ],
        out_specs=[
            pl.BlockSpec((TILE, 1), lambda i: (i, 0)),
            pl.BlockSpec((1, EMB_DIM, TILE), lambda i: (i // 2, 0, i % 2)),
            pl.BlockSpec((1, NUM_EMBS), lambda i: (0, 0)),
            pl.BlockSpec((1, 1), lambda i: (0, 0)),
            pl.BlockSpec((1, 1), lambda i: (0, 0)),
        ],
        out_shape=[
            jax.ShapeDtypeStruct((N_TOKENS, 1), jnp.int32),
            jax.ShapeDtypeStruct((B, C, H * Wd), jnp.float32),
            jax.ShapeDtypeStruct((1, NUM_EMBS), jnp.float32),
            jax.ShapeDtypeStruct((1, 1), jnp.float32),
            jax.ShapeDtypeStruct((1, 1), jnp.float32),
        ],
    )(x_flat, sx, swr, w_rev, wneg2r)

    return (loss[0, 0], zq.reshape(B, C, H, Wd), perp[0, 0], idx)


# speedup vs baseline: 1.2928x; 1.0520x over previous
import jax
import jax.numpy as jnp
from jax.experimental import pallas as pl

NUM_EMBS = 8192
EMB_DIM = 32
BETA = 0.25
N_TOKENS = 8192
TILE = 512
GRID = N_TOKENS // TILE
NCHUNK = 16
CW = NUM_EMBS // NCHUNK


def _body(x_ref, sx_ref, sw_ref, w_ref, wneg2_ref,
          idx_ref, zq_ref, hist_ref, loss_ref, perp_ref):
    step = pl.program_id(0)

    @pl.when(step == 0)
    def _init():
        hist_ref[...] = jnp.zeros_like(hist_ref)
        loss_ref[...] = jnp.zeros_like(loss_ref)
        perp_ref[...] = jnp.zeros_like(perp_ref)

    x = x_ref[...]

    mm2 = jax.lax.dot_general(x, wneg2_ref[...], (((1,), (1,)), ((), ())),
                              preferred_element_type=jnp.float32)
    d = (sx_ref[...] + sw_ref[...]) + mm2

    acc_v = d[:, 0:CW]
    acc_c = jnp.zeros((TILE, CW), jnp.int32)
    for c in range(1, NCHUNK):
        dc = d[:, c * CW:(c + 1) * CW]
        m = dc < acc_v
        acc_v = jnp.where(m, dc, acc_v)
        acc_c = jnp.where(m, c, acc_c)
    gmin = jnp.min(acc_v, axis=1, keepdims=True)
    scol = jax.lax.broadcasted_iota(jnp.int32, (TILE, CW), 1)
    jfull = acc_c * CW + scol
    cand = jnp.where(acc_v == gmin, jfull, NUM_EMBS)
    idx = jnp.min(cand, axis=1, keepdims=True)
    idx_ref[...] = idx

    col = jax.lax.broadcasted_iota(jnp.int32, (TILE, NUM_EMBS), 1)
    oh = (col == idx).astype(jnp.float32)
    q = jax.lax.dot_general(oh, w_ref[...], (((1,), (0,)), ((), ())),
                            preferred_element_type=jnp.float32)

    hist_ref[...] += jnp.sum(oh, axis=0)[None, :]
    zq = x + (q - x)
    zq_ref[...] = zq.T[None]
    loss_ref[...] += jnp.sum((q - x) ** 2)

    @pl.when(step == GRID - 1)
    def _fini():
        loss_ref[...] = (1.0 + BETA) * loss_ref[...] / (N_TOKENS * EMB_DIM)
        probs = hist_ref[...] / N_TOKENS
        ent = -jnp.sum(probs * jnp.log(probs + 1e-10))
        perp_ref[...] = jnp.exp(ent) * jnp.ones_like(perp_ref)


def kernel(z_e_x, W):
    B, C, H, Wd = z_e_x.shape
    x_flat = jnp.transpose(z_e_x, (0, 2, 3, 1)).reshape(-1, EMB_DIM)
    sx = jnp.sum(x_flat ** 2, axis=1, keepdims=True)
    sw = jnp.sum(W ** 2, axis=1)[None, :]

    idx, zq, hist, loss, perp = pl.pallas_call(
        _body,
        grid=(GRID,),
        in_specs=[
            pl.BlockSpec((TILE, EMB_DIM), lambda i: (i, 0)),
            pl.BlockSpec((TILE, 1), lambda i: (i, 0)),
            pl.BlockSpec((1, NUM_EMBS), lambda i: (0, 0)),
            pl.BlockSpec((NUM_EMBS, EMB_DIM), lambda i: (0, 0)),
            pl.BlockSpec((NUM_EMBS, EMB_DIM), lambda i: (0, 0)),
        ],
        out_specs=[
            pl.BlockSpec((TILE, 1), lambda i: (i, 0)),
            pl.BlockSpec((1, EMB_DIM, TILE), lambda i: (i // 2, 0, i % 2)),
            pl.BlockSpec((1, NUM_EMBS), lambda i: (0, 0)),
            pl.BlockSpec((1, 1), lambda i: (0, 0)),
            pl.BlockSpec((1, 1), lambda i: (0, 0)),
        ],
        out_shape=[
            jax.ShapeDtypeStruct((N_TOKENS, 1), jnp.int32),
            jax.ShapeDtypeStruct((B, C, H * Wd), jnp.float32),
            jax.ShapeDtypeStruct((1, NUM_EMBS), jnp.float32),
            jax.ShapeDtypeStruct((1, 1), jnp.float32),
            jax.ShapeDtypeStruct((1, 1), jnp.float32),
        ],
    )(x_flat, sx, sw, W, -2.0 * W)

    return (loss[0, 0], zq.reshape(B, C, H, Wd), perp[0, 0], idx)


# speedup vs baseline: 1.5225x; 1.1777x over previous
import jax
import jax.numpy as jnp
from jax.experimental import pallas as pl

NUM_EMBS = 8192
EMB_DIM = 32
BETA = 0.25
N_TOKENS = 8192
TILE = 512
GRID = N_TOKENS // TILE
NCHUNK = 16
CW = NUM_EMBS // NCHUNK


def _body(x_ref, sx_ref, sw_ref, wresh_ref, wneg2_ref,
          idx_ref, zq_ref, hist_ref, loss_ref, perp_ref):
    step = pl.program_id(0)

    @pl.when(step == 0)
    def _init():
        hist_ref[...] = jnp.zeros_like(hist_ref)
        loss_ref[...] = jnp.zeros_like(loss_ref)
        perp_ref[...] = jnp.zeros_like(perp_ref)

    x = x_ref[...]

    mm2 = jax.lax.dot_general(x, wneg2_ref[...], (((1,), (1,)), ((), ())),
                              preferred_element_type=jnp.float32)
    d = (sx_ref[...] + sw_ref[...]) + mm2

    acc_v = d[:, 0:CW]
    acc_c = jnp.zeros((TILE, CW), jnp.int32)
    for c in range(1, NCHUNK):
        dc = d[:, c * CW:(c + 1) * CW]
        m = dc < acc_v
        acc_v = jnp.where(m, dc, acc_v)
        acc_c = jnp.where(m, c, acc_c)
    gmin = jnp.min(acc_v, axis=1, keepdims=True)
    scol = jax.lax.broadcasted_iota(jnp.int32, (TILE, CW), 1)
    jfull = acc_c * CW + scol
    cand = jnp.where(acc_v == gmin, jfull, NUM_EMBS)
    idx = jnp.min(cand, axis=1, keepdims=True)
    idx_ref[...] = idx

    c_star = idx >> 9
    s_star = idx & (CW - 1)
    crow = jax.lax.broadcasted_iota(jnp.int32, (TILE, NCHUNK), 1)
    oh1 = (crow == c_star).astype(jnp.float32)
    oh2 = (scol == s_star).astype(jnp.float32)
    g = jax.lax.dot_general(oh2, wresh_ref[...], (((1,), (0,)), ((), ())),
                            preferred_element_type=jnp.float32)
    q = jnp.zeros((TILE, EMB_DIM), jnp.float32)
    for c in range(NCHUNK):
        q = q + oh1[:, c:c + 1] * g[:, c * EMB_DIM:(c + 1) * EMB_DIM]

    hist_ref[...] += jax.lax.dot_general(
        oh1, oh2, (((0,), (0,)), ((), ())),
        preferred_element_type=jnp.float32)
    zq = x + (q - x)
    zq_ref[...] = zq.T[None]
    loss_ref[...] += jnp.sum((q - x) ** 2)

    @pl.when(step == GRID - 1)
    def _fini():
        loss_ref[...] = (1.0 + BETA) * loss_ref[...] / (N_TOKENS * EMB_DIM)
        probs = hist_ref[...] / N_TOKENS
        ent = -jnp.sum(probs * jnp.log(probs + 1e-10))
        perp_ref[...] = jnp.exp(ent) * jnp.ones_like(perp_ref)


def kernel(z_e_x, W):
    B, C, H, Wd = z_e_x.shape
    x_flat = jnp.transpose(z_e_x, (0, 2, 3, 1)).reshape(-1, EMB_DIM)
    sx = jnp.sum(x_flat ** 2, axis=1, keepdims=True)
    sw = jnp.sum(W ** 2, axis=1)[None, :]
    wresh = jnp.transpose(W.reshape(NCHUNK, CW, EMB_DIM), (1, 0, 2))\
        .reshape(CW, NCHUNK * EMB_DIM)

    idx, zq, hist, loss, perp = pl.pallas_call(
        _body,
        grid=(GRID,),
        in_specs=[
            pl.BlockSpec((TILE, EMB_DIM), lambda i: (i, 0)),
            pl.BlockSpec((TILE, 1), lambda i: (i, 0)),
            pl.BlockSpec((1, NUM_EMBS), lambda i: (0, 0)),
            pl.BlockSpec((CW, NCHUNK * EMB_DIM), lambda i: (0, 0)),
            pl.BlockSpec((NUM_EMBS, EMB_DIM), lambda i: (0, 0)),
        ],
        out_specs=[
            pl.BlockSpec((TILE, 1), lambda i: (i, 0)),
            pl.BlockSpec((1, EMB_DIM, TILE), lambda i: (i // 2, 0, i % 2)),
            pl.BlockSpec((NCHUNK, CW), lambda i: (0, 0)),
            pl.BlockSpec((1, 1), lambda i: (0, 0)),
            pl.BlockSpec((1, 1), lambda i: (0, 0)),
        ],
        out_shape=[
            jax.ShapeDtypeStruct((N_TOKENS, 1), jnp.int32),
            jax.ShapeDtypeStruct((B, C, H * Wd), jnp.float32),
            jax.ShapeDtypeStruct((NCHUNK, CW), jnp.float32),
            jax.ShapeDtypeStruct((1, 1), jnp.float32),
            jax.ShapeDtypeStruct((1, 1), jnp.float32),
        ],
    )(x_flat, sx, sw, wresh, -2.0 * W)

    return (loss[0, 0], zq.reshape(B, C, H, Wd), perp[0, 0], idx)


# speedup vs baseline: 1.5308x; 1.0055x over previous
import jax
import jax.numpy as jnp
from jax.experimental import pallas as pl

NUM_EMBS = 8192
EMB_DIM = 32
BETA = 0.25
N_TOKENS = 8192
TILE = 1024
GRID = N_TOKENS // TILE
NCHUNK = 16
CW = NUM_EMBS // NCHUNK


def _body(x_ref, sx_ref, sw_ref, wresh_ref, wneg2_ref,
          idx_ref, zq_ref, hist_ref, loss_ref, perp_ref):
    step = pl.program_id(0)

    @pl.when(step == 0)
    def _init():
        hist_ref[...] = jnp.zeros_like(hist_ref)
        loss_ref[...] = jnp.zeros_like(loss_ref)
        perp_ref[...] = jnp.zeros_like(perp_ref)

    x = x_ref[...]

    mm2 = jax.lax.dot_general(x, wneg2_ref[...], (((1,), (1,)), ((), ())),
                              preferred_element_type=jnp.float32)
    d = (sx_ref[...] + sw_ref[...]) + mm2

    acc_v = d[:, 0:CW]
    acc_c = jnp.zeros((TILE, CW), jnp.int32)
    for c in range(1, NCHUNK):
        dc = d[:, c * CW:(c + 1) * CW]
        m = dc < acc_v
        acc_v = jnp.where(m, dc, acc_v)
        acc_c = jnp.where(m, c, acc_c)
    gmin = jnp.min(acc_v, axis=1, keepdims=True)
    scol = jax.lax.broadcasted_iota(jnp.int32, (TILE, CW), 1)
    jfull = acc_c * CW + scol
    cand = jnp.where(acc_v == gmin, jfull, NUM_EMBS)
    idx = jnp.min(cand, axis=1, keepdims=True)
    idx_ref[...] = idx

    c_star = idx >> 9
    s_star = idx & (CW - 1)
    crow = jax.lax.broadcasted_iota(jnp.int32, (TILE, NCHUNK), 1)
    oh1 = (crow == c_star).astype(jnp.float32)
    oh2 = (scol == s_star).astype(jnp.float32)
    g = jax.lax.dot_general(oh2, wresh_ref[...], (((1,), (0,)), ((), ())),
                            preferred_element_type=jnp.float32)
    q = jnp.zeros((TILE, EMB_DIM), jnp.float32)
    for c in range(NCHUNK):
        q = q + oh1[:, c:c + 1] * g[:, c * EMB_DIM:(c + 1) * EMB_DIM]

    hist_ref[...] += jax.lax.dot_general(
        oh1, oh2, (((0,), (0,)), ((), ())),
        preferred_element_type=jnp.float32)
    zq = x + (q - x)
    zq_ref[...] = zq.T[None]
    loss_ref[...] += jnp.sum((q - x) ** 2)

    @pl.when(step == GRID - 1)
    def _fini():
        loss_ref[...] = (1.0 + BETA) * loss_ref[...] / (N_TOKENS * EMB_DIM)
        probs = hist_ref[...] / N_TOKENS
        ent = -jnp.sum(probs * jnp.log(probs + 1e-10))
        perp_ref[...] = jnp.exp(ent) * jnp.ones_like(perp_ref)


def kernel(z_e_x, W):
    B, C, H, Wd = z_e_x.shape
    x_flat = jnp.transpose(z_e_x, (0, 2, 3, 1)).reshape(-1, EMB_DIM)
    sx = jnp.sum(x_flat ** 2, axis=1, keepdims=True)
    sw = jnp.sum(W ** 2, axis=1)[None, :]
    wresh = jnp.transpose(W.reshape(NCHUNK, CW, EMB_DIM), (1, 0, 2))\
        .reshape(CW, NCHUNK * EMB_DIM)

    idx, zq, hist, loss, perp = pl.pallas_call(
        _body,
        grid=(GRID,),
        in_specs=[
            pl.BlockSpec((TILE, EMB_DIM), lambda i: (i, 0)),
            pl.BlockSpec((TILE, 1), lambda i: (i, 0)),
            pl.BlockSpec((1, NUM_EMBS), lambda i: (0, 0)),
            pl.BlockSpec((CW, NCHUNK * EMB_DIM), lambda i: (0, 0)),
            pl.BlockSpec((NUM_EMBS, EMB_DIM), lambda i: (0, 0)),
        ],
        out_specs=[
            pl.BlockSpec((TILE, 1), lambda i: (i, 0)),
            pl.BlockSpec((1, EMB_DIM, TILE), lambda i: (i, 0, 0)),
            pl.BlockSpec((NCHUNK, CW), lambda i: (0, 0)),
            pl.BlockSpec((1, 1), lambda i: (0, 0)),
            pl.BlockSpec((1, 1), lambda i: (0, 0)),
        ],
        out_shape=[
            jax.ShapeDtypeStruct((N_TOKENS, 1), jnp.int32),
            jax.ShapeDtypeStruct((B, C, H * Wd), jnp.float32),
            jax.ShapeDtypeStruct((NCHUNK, CW), jnp.float32),
            jax.ShapeDtypeStruct((1, 1), jnp.float32),
            jax.ShapeDtypeStruct((1, 1), jnp.float32),
        ],
    )(x_flat, sx, sw, wresh, -2.0 * W)

    return (loss[0, 0], zq.reshape(B, C, H, Wd), perp[0, 0], idx)
